# baseline (speedup 1.0000x reference)
import functools
import math

import jax
import jax.numpy as jnp
from jax.experimental import pallas as pl

S, B, H, NH = 2048, 1, 1024, 16
DH = H // NH
BQ = 256
NQ = S // BQ
BS = 256


def _proj_kernel(x_ref, w_ref, b_ref, o_ref):
    x = x_ref[...]
    w = w_ref[...]
    acc = jax.lax.dot_general(
        x, w, (((1,), (1,)), ((), ())),
        preferred_element_type=jnp.float32,
        precision=jax.lax.Precision.HIGHEST)
    o_ref[...] = acc + b_ref[...]


def _attn_kernel(q_ref, k_ref, v_ref, m_ref, o_ref, *, scale):
    q = q_ref[0]
    k = k_ref[0]
    v = v_ref[0]
    s = jax.lax.dot_general(
        q, k, (((1,), (1,)), ((), ())),
        preferred_element_type=jnp.float32,
        precision=jax.lax.Precision.HIGHEST) * scale
    msk = m_ref[...] > 0
    s = jnp.where(msk, s, -1e9)
    mx = jnp.max(s, axis=1, keepdims=True)
    p = jnp.where(msk, jnp.exp(s - mx), 0.0)
    l = jnp.sum(p, axis=1, keepdims=True)
    ctx = jax.lax.dot_general(
        p, v, (((1,), (0,)), ((), ())),
        preferred_element_type=jnp.float32,
        precision=jax.lax.Precision.HIGHEST)
    o_ref[0] = jnp.where(l > 0, ctx / jnp.where(l > 0, l, 1.0), 0.0)


def kernel(hidden_states, attention_mask, Wq, bq, Wk, bk, Wv, bv):
    x = hidden_states.reshape(S, H)
    w_all = jnp.concatenate([Wq, Wk, Wv], axis=0)
    b_all = jnp.concatenate([bq, bk, bv], axis=0).reshape(1, 3 * H)

    qkv = pl.pallas_call(
        _proj_kernel,
        grid=(S // BS,),
        in_specs=[
            pl.BlockSpec((BS, H), lambda i: (i, 0)),
            pl.BlockSpec((3 * H, H), lambda i: (0, 0)),
            pl.BlockSpec((1, 3 * H), lambda i: (0, 0)),
        ],
        out_specs=pl.BlockSpec((BS, 3 * H), lambda i: (i, 0)),
        out_shape=jax.ShapeDtypeStruct((S, 3 * H), jnp.float32),
    )(x, w_all, b_all)

    qkv = qkv.reshape(S, 3, NH, DH).transpose(1, 2, 0, 3)
    q, k, v = qkv[0], qkv[1], qkv[2]

    scale = 1.0 / math.sqrt(DH)
    ctx = pl.pallas_call(
        functools.partial(_attn_kernel, scale=scale),
        grid=(NQ, NH),
        in_specs=[
            pl.BlockSpec((1, BQ, DH), lambda i, h: (h, i, 0)),
            pl.BlockSpec((1, S, DH), lambda i, h: (h, 0, 0)),
            pl.BlockSpec((1, S, DH), lambda i, h: (h, 0, 0)),
            pl.BlockSpec((BQ, S), lambda i, h: (i, 0)),
        ],
        out_specs=pl.BlockSpec((1, BQ, DH), lambda i, h: (h, i, 0)),
        out_shape=jax.ShapeDtypeStruct((NH, S, DH), jnp.float32),
    )(q, k, v, attention_mask)

    return ctx.transpose(1, 0, 2).reshape(S, B, H)


# speedup vs baseline: 2.8794x; 2.8794x over previous
import functools
import math

import jax
import jax.numpy as jnp
from jax.experimental import pallas as pl

S, B, H, NH = 2048, 1, 1024, 16
DH = H // NH
BQ = 256
NQ = S // BQ
BS = 256


def _proj_kernel(x_ref, w_ref, b_ref, o_ref):
    x = x_ref[...]
    w = w_ref[...]
    acc = jax.lax.dot_general(
        x, w, (((1,), (1,)), ((), ())),
        preferred_element_type=jnp.float32,
        precision=jax.lax.Precision.DEFAULT)
    o_ref[...] = acc + b_ref[...]


def _attn_kernel(q_ref, k_ref, v_ref, m_ref, o_ref, *, scale):
    q = q_ref[0]
    k = k_ref[0]
    v = v_ref[0]
    s = jax.lax.dot_general(
        q, k, (((1,), (1,)), ((), ())),
        preferred_element_type=jnp.float32,
        precision=jax.lax.Precision.DEFAULT) * scale
    msk = m_ref[...] > 0
    s = jnp.where(msk, s, -1e9)
    mx = jnp.max(s, axis=1, keepdims=True)
    p = jnp.where(msk, jnp.exp(s - mx), 0.0)
    l = jnp.sum(p, axis=1, keepdims=True)
    ctx = jax.lax.dot_general(
        p, v, (((1,), (0,)), ((), ())),
        preferred_element_type=jnp.float32,
        precision=jax.lax.Precision.DEFAULT)
    o_ref[0] = jnp.where(l > 0, ctx / jnp.where(l > 0, l, 1.0), 0.0)


def kernel(hidden_states, attention_mask, Wq, bq, Wk, bk, Wv, bv):
    x = hidden_states.reshape(S, H)
    w_all = jnp.concatenate([Wq, Wk, Wv], axis=0)
    b_all = jnp.concatenate([bq, bk, bv], axis=0).reshape(1, 3 * H)

    qkv = pl.pallas_call(
        _proj_kernel,
        grid=(S // BS,),
        in_specs=[
            pl.BlockSpec((BS, H), lambda i: (i, 0)),
            pl.BlockSpec((3 * H, H), lambda i: (0, 0)),
            pl.BlockSpec((1, 3 * H), lambda i: (0, 0)),
        ],
        out_specs=pl.BlockSpec((BS, 3 * H), lambda i: (i, 0)),
        out_shape=jax.ShapeDtypeStruct((S, 3 * H), jnp.float32),
    )(x, w_all, b_all)

    qkv = qkv.reshape(S, 3, NH, DH).transpose(1, 2, 0, 3)
    q, k, v = qkv[0], qkv[1], qkv[2]

    scale = 1.0 / math.sqrt(DH)
    ctx = pl.pallas_call(
        functools.partial(_attn_kernel, scale=scale),
        grid=(NQ, NH),
        in_specs=[
            pl.BlockSpec((1, BQ, DH), lambda i, h: (h, i, 0)),
            pl.BlockSpec((1, S, DH), lambda i, h: (h, 0, 0)),
            pl.BlockSpec((1, S, DH), lambda i, h: (h, 0, 0)),
            pl.BlockSpec((BQ, S), lambda i, h: (i, 0)),
        ],
        out_specs=pl.BlockSpec((1, BQ, DH), lambda i, h: (h, i, 0)),
        out_shape=jax.ShapeDtypeStruct((NH, S, DH), jnp.float32),
    )(q, k, v, attention_mask)

    return ctx.transpose(1, 0, 2).reshape(S, B, H)


# speedup vs baseline: 5.1289x; 1.7812x over previous
import functools
import math

import jax
import jax.numpy as jnp
from jax.experimental import pallas as pl
from jax.experimental.pallas import tpu as pltpu

S, B, H, NH = 2048, 1, 1024, 16
DH = H // NH
BQ = 256
NQ = S // BQ
NP = NH // 2
NEG = -1e9


def _proj_kernel(x_ref, w_ref, b_ref, q_ref, k_ref, v_ref):
    acc = jax.lax.dot_general(
        x_ref[...], w_ref[0], (((1,), (1,)), ((), ())),
        preferred_element_type=jnp.float32) + b_ref[0]
    acc = acc.astype(jnp.bfloat16)
    q_ref[0] = acc[:, 0:64]
    k_ref[0] = acc[:, 64:128]
    v_ref[0] = acc[:, 128:192]
    q_ref[1] = acc[:, 192:256]
    k_ref[1] = acc[:, 256:320]
    v_ref[1] = acc[:, 320:384]


def _attn_kernel(q_ref, k_ref, v_ref, m_ref, o_ref, bias_scr):
    p_id = pl.program_id(0)
    i = pl.program_id(1)

    @pl.when(jnp.logical_and(p_id == 0, i == 0))
    def _():
        bias_scr[...] = jnp.where(
            m_ref[...] > 0, 0.0, NEG).astype(jnp.bfloat16)

    q = q_ref[...]
    k = k_ref[...]
    v = v_ref[...]
    s = jax.lax.dot_general(
        q, k, (((2,), (2,)), ((0,), (0,))),
        preferred_element_type=jnp.float32)
    bias = bias_scr[pl.ds(i * BQ, BQ), :]
    s = s + bias.astype(jnp.float32)[None]
    mx = jnp.max(s, axis=2, keepdims=True)
    p = jnp.exp(s - mx)
    l = jnp.sum(p, axis=2, keepdims=True)
    ctx = jax.lax.dot_general(
        p, v, (((2,), (1,)), ((0,), (0,))),
        preferred_element_type=jnp.float32)
    ctx = jnp.where(mx > -5e8, ctx / l, 0.0)
    o_ref[...] = ctx.transpose(1, 0, 2).reshape(BQ, 2 * DH)


def kernel(hidden_states, attention_mask, Wq, bq, Wk, bk, Wv, bv):
    x = hidden_states.reshape(S, H)
    scale = 1.0 / math.sqrt(DH)
    w = jnp.stack([Wq * scale, Wk, Wv], 0).reshape(3, NH, DH, H)
    w = w.transpose(1, 0, 2, 3).reshape(NP, 6 * DH, H)
    b = jnp.stack([bq * scale, bk, bv], 0).reshape(3, NH, DH)
    b = b.transpose(1, 0, 2).reshape(NP, 1, 6 * DH)

    q, k, v = pl.pallas_call(
        _proj_kernel,
        grid=(NP,),
        in_specs=[
            pl.BlockSpec((S, H), lambda p: (0, 0)),
            pl.BlockSpec((1, 6 * DH, H), lambda p: (p, 0, 0)),
            pl.BlockSpec((1, 1, 6 * DH), lambda p: (p, 0, 0)),
        ],
        out_specs=[
            pl.BlockSpec((2, S, DH), lambda p: (p, 0, 0)),
            pl.BlockSpec((2, S, DH), lambda p: (p, 0, 0)),
            pl.BlockSpec((2, S, DH), lambda p: (p, 0, 0)),
        ],
        out_shape=[jax.ShapeDtypeStruct((NH, S, DH), jnp.bfloat16)] * 3,
    )(x, w, b)

    ctx = pl.pallas_call(
        _attn_kernel,
        grid=(NP, NQ),
        in_specs=[
            pl.BlockSpec((2, BQ, DH), lambda p, i: (p, i, 0)),
            pl.BlockSpec((2, S, DH), lambda p, i: (p, 0, 0)),
            pl.BlockSpec((2, S, DH), lambda p, i: (p, 0, 0)),
            pl.BlockSpec((S, S), lambda p, i: (0, 0)),
        ],
        out_specs=pl.BlockSpec((BQ, 2 * DH), lambda p, i: (i, p)),
        out_shape=jax.ShapeDtypeStruct((S, H), jnp.float32),
        scratch_shapes=[pltpu.VMEM((S, S), jnp.bfloat16)],
    )(q, k, v, attention_mask)

    return ctx.reshape(S, B, H)
